# baseline (speedup 1.0000x reference)
import functools

import jax
import jax.numpy as jnp
from jax import lax
from jax.experimental import pallas as pl
from jax.experimental.pallas import tpu as pltpu
from jax.experimental.pallas import tpu_sc as plsc

NUM_H = 4
KV = 513
NE = 257
DIM = 64
ROWS = NUM_H * KV
NC, NS, L = 2, 16, 16
NW = NC * NS

RB = 2
SEQ = 200
C = RB * SEQ
JB = 4
JW = C // JB


def _t2_body(ww_ref, we_ref, t2_ref):
    r = lax.broadcasted_iota(jnp.int32, (ROWS, NE), 0)
    c = lax.broadcasted_iota(jnp.int32, (ROWS, NE), 1)
    e = lax.rem(r, KV) // 2
    onehot = jnp.where(c == e, 1.0, 0.0)
    emb = jnp.dot(onehot, we_ref[...], preferred_element_type=jnp.float32)
    t2_ref[...] = emb * ww_ref[...]


def _build_t2(W_weights, W_emb):
    return pl.pallas_call(
        _t2_body,
        out_shape=jax.ShapeDtypeStruct((ROWS, DIM), jnp.float32),
    )(W_weights, W_emb)


@functools.lru_cache(maxsize=None)
def _make_sc_lookup(B):
    RPW = B // NW
    NCHUNK = RPW // RB

    mesh = plsc.VectorSubcoreMesh(core_axis_name="c", subcore_axis_name="s")

    @functools.partial(
        pl.kernel,
        out_type=jax.ShapeDtypeStruct((B, SEQ, DIM), jnp.float32),
        mesh=mesh,
        scratch_types=[
            pltpu.VMEM((RB, SEQ, NUM_H), jnp.int32),
            pltpu.VMEM((NUM_H, JB, JW), jnp.int32),
            pltpu.VMEM((RB, SEQ, DIM), jnp.float32),
            pltpu.SemaphoreType.DMA,
        ],
        compiler_params=pltpu.CompilerParams(
            needs_layout_passes=False, use_tc_tiling_on_sc=False
        ),
    )
    def sc_lookup(x_hbm, t2_hbm, out_hbm, x_v, widx_v, out_v, sem):
        cid = lax.axis_index("c")
        sid = lax.axis_index("s")
        wid = sid * NC + cid
        base = wid * RPW
        iota = lax.iota(jnp.int32, 16)

        def chunk(ci, carry):
            r0 = base + ci * RB
            pltpu.sync_copy(x_hbm.at[pl.ds(r0, RB)], x_v)

            for h in range(NUM_H):
                hv = jnp.full((16,), h, jnp.int32)
                def idx_body(t, _, h=h, hv=hv):
                    p = iota + t * 16
                    v = plsc.load_gather(
                        x_v, [p // SEQ, lax.rem(p, SEQ), hv]
                    )
                    plsc.store_scatter(
                        widx_v,
                        [hv, p // JW, lax.rem(p, JW)],
                        v + jnp.int32(KV * h),
                    )
                    return _
                lax.fori_loop(0, C // 16, idx_body, 0, unroll=True)

            first = [
                pltpu.async_copy(
                    t2_hbm.at[widx_v.at[0, j]],
                    out_v.at[j // 2, pl.ds((j % 2) * JW, JW)],
                    sem,
                )
                for j in range(JB)
            ]
            for d in first:
                d.wait()
            adds = [
                pltpu.async_copy(
                    t2_hbm.at[widx_v.at[h, j]],
                    out_v.at[j // 2, pl.ds((j % 2) * JW, JW)],
                    sem,
                    add=True,
                )
                for h in range(1, NUM_H)
                for j in range(JB)
            ]
            for d in adds:
                d.wait()

            pltpu.sync_copy(out_v, out_hbm.at[pl.ds(r0, RB)])
            return carry

        lax.fori_loop(0, NCHUNK, chunk, 0)

    return sc_lookup


def kernel(x, W_weights, W_emb):
    B, S, H = x.shape
    t2 = _build_t2(W_weights, W_emb)
    return _make_sc_lookup(B)(x.astype(jnp.int32), t2)


# speedup vs baseline: 4.2306x; 4.2306x over previous
import functools

import jax
import jax.numpy as jnp
from jax import lax
from jax.experimental import pallas as pl
from jax.experimental.pallas import tpu as pltpu
from jax.experimental.pallas import tpu_sc as plsc

NUM_H = 4
KV = 513
NE = 257
DIM = 64
ROWS = NUM_H * KV
NC, NS, L = 2, 16, 16
NW = NC * NS

NDB = 8
DB = DIM // NDB
NBQ = NW // NDB
SG = 4


def _t2_body(ww_ref, we_ref, t2_ref):
    r = lax.broadcasted_iota(jnp.int32, (ROWS, NE), 0)
    c = lax.broadcasted_iota(jnp.int32, (ROWS, NE), 1)
    e = lax.rem(r, KV) // 2
    onehot = jnp.where(c == e, 1.0, 0.0)
    emb = jnp.dot(onehot, we_ref[...], preferred_element_type=jnp.float32)
    t2 = emb * ww_ref[...]
    t2b = t2.reshape(ROWS, NDB, DB).transpose(1, 0, 2)
    u = lax.bitcast_convert_type(
        t2b.astype(jnp.bfloat16), jnp.uint16
    ).astype(jnp.uint32)
    packed = u[:, :, DB // 2:] << 16 | u[:, :, : DB // 2]
    padded = jnp.pad(packed, ((0, 0), (0, 0), (0, 1)))
    t2_ref[...] = lax.bitcast_convert_type(padded, jnp.int32)


def _build_t2(W_weights, W_emb):
    return pl.pallas_call(
        _t2_body,
        out_shape=jax.ShapeDtypeStruct((NDB, ROWS, DB // 2 + 1), jnp.int32),
    )(W_weights, W_emb)


@functools.lru_cache(maxsize=None)
def _make_sc_lookup(B, S):
    BQ = B // NBQ
    NBB = BQ // 128
    NSC = S // SG

    mesh = plsc.VectorSubcoreMesh(core_axis_name="c", subcore_axis_name="s")

    @functools.partial(
        pl.kernel,
        out_type=jax.ShapeDtypeStruct((S, NDB, B // 128, DB, 128), jnp.float32),
        mesh=mesh,
        scratch_types=[
            pltpu.VMEM((ROWS, DB // 2 + 1), jnp.int32),
            pltpu.VMEM((2, SG, NUM_H, BQ), jnp.int32),
            pltpu.VMEM((2, SG, BQ // 128, DB, 128), jnp.float32),
            pltpu.SemaphoreType.DMA,
            pltpu.SemaphoreType.DMA,
        ],
        compiler_params=pltpu.CompilerParams(
            needs_layout_passes=False, use_tc_tiling_on_sc=False
        ),
    )
    def sc_lookup(xt_hbm, t2_hbm, out_hbm, t2_v, x_v, out_v, sem_x, sem_o):
        cid = lax.axis_index("c")
        sid = lax.axis_index("s")
        wid = sid * NC + cid
        dblk = lax.rem(wid, NDB)
        bq = wid // NDB
        b0 = bq * BQ
        bb0 = bq * NBB

        def x_src(ci):
            return xt_hbm.at[pl.ds(ci * SG, SG), :, pl.ds(b0, BQ)]

        def out_dst(ci):
            return out_hbm.at[pl.ds(ci * SG, SG), dblk, pl.ds(bb0, NBB)]

        pltpu.sync_copy(t2_hbm.at[dblk], t2_v)
        pltpu.async_copy(x_src(0), x_v.at[0], sem_x)

        def chunk(ci, carry):
            buf = lax.rem(ci, 2)
            pltpu.make_async_copy(x_src(ci), x_v.at[buf], sem_x).wait()

            @pl.when(ci < NSC - 1)
            def _prefetch():
                pltpu.async_copy(x_src(ci + 1), x_v.at[1 - buf], sem_x)

            @pl.when(ci >= 2)
            def _drain_out():
                pltpu.make_async_copy(
                    out_v.at[buf], out_dst(ci - 2), sem_o
                ).wait()

            for sg in range(SG):
                @plsc.parallel_loop(0, BQ // 16, unroll=4)
                def group(g, sg=sg):
                    col = g * 16
                    w0 = x_v[buf, sg, 0, pl.ds(col, 16)]
                    w1 = x_v[buf, sg, 1, pl.ds(col, 16)] + jnp.int32(KV)
                    w2 = x_v[buf, sg, 2, pl.ds(col, 16)] + jnp.int32(2 * KV)
                    w3 = x_v[buf, sg, 3, pl.ds(col, 16)] + jnp.int32(3 * KV)
                    hmask = jnp.full((16,), -65536, jnp.int32)
                    for dp in range(DB // 2):
                        dv = jnp.full((16,), dp, jnp.int32)
                        g0 = plsc.load_gather(t2_v, [w0, dv])
                        g1 = plsc.load_gather(t2_v, [w1, dv])
                        g2 = plsc.load_gather(t2_v, [w2, dv])
                        g3 = plsc.load_gather(t2_v, [w3, dv])
                        acc = (
                            (plsc.bitcast(g0, jnp.bfloat16)
                             + plsc.bitcast(g1, jnp.bfloat16))
                            + (plsc.bitcast(g2, jnp.bfloat16)
                               + plsc.bitcast(g3, jnp.bfloat16))
                        )
                        acci = plsc.bitcast(acc, jnp.int32)
                        acc_lo = plsc.bitcast(acci << 16, jnp.float32)
                        acc_hi = plsc.bitcast(acci & hmask, jnp.float32)
                        cs = pl.ds(lax.rem(g, 8) * 16, 16)
                        out_v[buf, sg, g // 8, dp, cs] = acc_lo
                        out_v[buf, sg, g // 8, dp + DB // 2, cs] = acc_hi

            pltpu.async_copy(out_v.at[buf], out_dst(ci), sem_o)
            return carry

        lax.fori_loop(0, NSC, chunk, 0)
        pltpu.make_async_copy(
            out_v.at[lax.rem(NSC - 2, 2)], out_dst(NSC - 2), sem_o
        ).wait()
        pltpu.make_async_copy(
            out_v.at[lax.rem(NSC - 1, 2)], out_dst(NSC - 1), sem_o
        ).wait()

    return sc_lookup


def kernel(x, W_weights, W_emb):
    B, S, H = x.shape
    t2 = _build_t2(W_weights, W_emb)
    xt = jnp.transpose(x.astype(jnp.int32), (1, 2, 0))
    y = _make_sc_lookup(B, S)(xt, t2)
    return y.transpose(2, 4, 0, 1, 3).reshape(B, S, DIM)
